# baseline (speedup 1.0000x reference)
import jax
import jax.numpy as jnp
from jax import lax
from jax.experimental import pallas as pl
from jax.experimental.pallas import tpu as pltpu
from jax.experimental.pallas import tpu_sc as plsc

_M = 100000
_D = 256
_B = 4096
_K = 99
_KP = 128
_MOM = 0.99
_MARGIN = 10.0

_NW = 32
_BPW = _B // _NW
_RCH = 32
_TM = 20000
_NT = _M // _TM


def _sc_gather_body(mem_h, labels_h, idx_h, rows_o, bl_o, win_o,
                    idx_all, tag, rows_buf, bl_buf, win_buf, sem):
    c = lax.axis_index("c")
    s = lax.axis_index("s")
    wid = s * 2 + c
    base = wid * _BPW

    @pl.when(wid == 0)
    def _():
        pltpu.sync_copy(idx_h, idx_all)

    @pl.when(wid != 0)
    def _():
        pltpu.sync_copy(idx_h.at[pl.ds(base, _BPW)],
                        idx_all.at[pl.ds(base, _BPW)])

    for r in range(_BPW // _RCH):
        off = base + r * _RCH
        pltpu.async_copy(mem_h.at[idx_all.at[pl.ds(off, _RCH)]],
                         rows_buf, sem).wait()
        pltpu.sync_copy(rows_buf, rows_o.at[pl.ds(off, _RCH)])

    pltpu.async_copy(labels_h.at[idx_all.at[pl.ds(base, _BPW)]],
                     bl_buf, sem).wait()
    pltpu.sync_copy(bl_buf, bl_o.at[pl.ds(base, _BPW)])

    @pl.when(wid == 0)
    def _():
        def scat(j, carry):
            iv = idx_all[pl.ds(j * 16, 16)]
            jv = lax.iota(jnp.int32, 16) + j * 16
            plsc.store_scatter(tag, (iv,), jv)
            return carry

        lax.fori_loop(0, _B // 16, scat, 0)

        def gath(j, carry):
            iv = idx_all[pl.ds(j * 16, 16)]
            jv = lax.iota(jnp.int32, 16) + j * 16
            t = plsc.load_gather(tag, (iv,))
            win_buf[pl.ds(j * 16, 16)] = jnp.where(t == jv, 1.0, 0.0)
            return carry

        lax.fori_loop(0, _B // 16, gath, 0)
        pltpu.sync_copy(win_buf, win_o)


def _sc_gather(mem, bank_labels, idx):
    mesh = plsc.VectorSubcoreMesh(core_axis_name="c", subcore_axis_name="s")
    f = pl.kernel(
        _sc_gather_body,
        out_type=[
            jax.ShapeDtypeStruct((_B, _D), jnp.float32),
            jax.ShapeDtypeStruct((_B,), jnp.int32),
            jax.ShapeDtypeStruct((_B,), jnp.float32),
        ],
        mesh=mesh,
        scratch_types=[
            pltpu.VMEM((_B,), jnp.int32),
            pltpu.VMEM((_M,), jnp.int32),
            pltpu.VMEM((_RCH, _D), jnp.float32),
            pltpu.VMEM((_BPW,), jnp.int32),
            pltpu.VMEM((_B,), jnp.float32),
            pltpu.SemaphoreType.DMA,
        ],
        compiler_params=pltpu.CompilerParams(needs_layout_passes=False),
    )
    return f(mem, bank_labels, idx)


def _tc_sweep_body(lab_ref, mem_ref, sums_ref, counts_ref):
    t = pl.program_id(0)

    @pl.when(t == 0)
    def _():
        sums_ref[...] = jnp.zeros_like(sums_ref)
        counts_ref[...] = jnp.zeros_like(counts_ref)

    lab = lab_ref[0].astype(jnp.bfloat16)
    kio = lax.broadcasted_iota(
        jnp.int32, (_KP, _TM), 0).astype(jnp.bfloat16)
    eq = lab == kio
    oh = jnp.where(eq, jnp.bfloat16(1.0), jnp.bfloat16(0.0))
    sums_ref[...] += jnp.dot(oh, mem_ref[...],
                             preferred_element_type=jnp.float32)
    counts_ref[...] += jnp.sum(oh, axis=1, keepdims=True,
                               dtype=jnp.float32)


def _tc_sweep(labels3, mem):
    return pl.pallas_call(
        _tc_sweep_body,
        grid=(_NT,),
        in_specs=[
            pl.BlockSpec((1, 1, _TM), lambda i: (i, 0, 0)),
            pl.BlockSpec((_TM, _D), lambda i: (i, 0)),
        ],
        out_specs=[
            pl.BlockSpec((_KP, _D), lambda i: (0, 0)),
            pl.BlockSpec((_KP, 1), lambda i: (0, 0)),
        ],
        out_shape=[
            jax.ShapeDtypeStruct((_KP, _D), jnp.float32),
            jax.ShapeDtypeStruct((_KP, 1), jnp.float32),
        ],
    )(labels3, mem)


_BS = 1024
_NS = _B // _BS


def _tc_epilogue_body(sums_ref, counts_ref, rows_ref, blg_ref, win_ref,
                      ilr_ref, x_ref, ilc_ref, proto_ref, out_ref,
                      s2, c2, ps, psq):
    s = pl.program_id(0)

    @pl.when(s == 0)
    def _():
        s2[...] = sums_ref[...]
        c2[...] = counts_ref[...]

    @pl.when(s < _NS)
    def _():
        kiob = lax.broadcasted_iota(jnp.int32, (_KP, _BS), 0)
        win = win_ref[...]
        oh_sub = jnp.where(blg_ref[...] == kiob, win, 0.0)
        oh_add = jnp.where(ilr_ref[...] == kiob, win, 0.0)
        s2[...] += (jnp.dot(oh_add, x_ref[...],
                            preferred_element_type=jnp.float32)
                    - jnp.dot(oh_sub, rows_ref[...],
                              preferred_element_type=jnp.float32))
        c2[...] += jnp.sum(oh_add - oh_sub, axis=1, keepdims=True)

    @pl.when(s == _NS)
    def _():
        proto_pad = jnp.concatenate(
            [proto_ref[...], jnp.zeros((_KP - _K, _D), jnp.float32)], axis=0)
        ps[...] = (_MOM * proto_pad
                   + (1.0 - _MOM) * (s2[...] / jnp.maximum(c2[...], 1.0)))
        ones_row = jnp.ones((1, _D), dtype=jnp.float32)
        psq[...] = lax.dot_general(ones_row, ps[...] * ps[...],
                                   (((1,), (1,)), ((), ())),
                                   preferred_element_type=jnp.float32)

    @pl.when(s >= _NS)
    def _():
        x = x_ref[...]
        xp = lax.dot_general(x, ps[...], (((1,), (1,)), ((), ())),
                             preferred_element_type=jnp.float32)
        x_sq = jnp.sum(x * x, axis=1, keepdims=True)
        d2 = x_sq + psq[...] - 2.0 * xp
        d = jnp.sqrt(jnp.maximum(d2, 1e-12))
        cio = lax.broadcasted_iota(jnp.int32, (_BS, _KP), 1)
        own = ilc_ref[...] == cio
        per = jnp.where(own, d, jnp.maximum(0.0, _MARGIN - d))
        per = jnp.where(cio < _K, per, 0.0)
        partial = jnp.sum(per) / (_B * _K)

        @pl.when(s == _NS)
        def _():
            out_ref[...] = partial * jnp.ones((1, 1), jnp.float32)

        @pl.when(s > _NS)
        def _():
            out_ref[...] += partial


def _tc_epilogue(sums, counts, rows, blg_r, win_r, il_r, x, il_c, proto):
    ilast = lambda s: (0, jnp.minimum(s, _NS - 1))
    xmap = lambda s: (jnp.where(s < _NS, s, 2 * _NS - 1 - s), 0)
    return pl.pallas_call(
        _tc_epilogue_body,
        grid=(2 * _NS,),
        in_specs=[
            pl.BlockSpec((_KP, _D), lambda s: (0, 0)),
            pl.BlockSpec((_KP, 1), lambda s: (0, 0)),
            pl.BlockSpec((_BS, _D), lambda s: (jnp.minimum(s, _NS - 1), 0)),
            pl.BlockSpec((1, _BS), ilast),
            pl.BlockSpec((1, _BS), ilast),
            pl.BlockSpec((1, _BS), ilast),
            pl.BlockSpec((_BS, _D), xmap),
            pl.BlockSpec((_BS, 1), xmap),
            pl.BlockSpec((_K, _D), lambda s: (0, 0)),
        ],
        out_specs=pl.BlockSpec((1, 1), lambda s: (0, 0)),
        out_shape=jax.ShapeDtypeStruct((1, 1), jnp.float32),
        scratch_shapes=[
            pltpu.VMEM((_KP, _D), jnp.float32),
            pltpu.VMEM((_KP, 1), jnp.float32),
            pltpu.VMEM((_KP, _D), jnp.float32),
            pltpu.VMEM((1, _KP), jnp.float32),
        ],
    )(sums, counts, rows, blg_r, win_r, il_r, x, il_c, proto)


def kernel(mem, bank_labels, idx, input_vectors, input_labels, prototypes):
    rows, blg, win = _sc_gather(mem, bank_labels, idx)
    labels3 = bank_labels.reshape(_NT, 1, _TM)
    sums, counts = _tc_sweep(labels3, mem)
    blg_r = blg.reshape(1, _B)
    win_r = win.reshape(1, _B)
    il_r = input_labels.reshape(1, _B)
    il_c = input_labels.reshape(_B, 1)
    out = _tc_epilogue(sums, counts, rows, blg_r, win_r, il_r,
                       input_vectors, il_c, prototypes)
    return out[0, 0]


# speedup vs baseline: 1.0805x; 1.0805x over previous
import jax
import jax.numpy as jnp
from jax import lax
from jax.experimental import pallas as pl
from jax.experimental.pallas import tpu as pltpu
from jax.experimental.pallas import tpu_sc as plsc

_M = 100000
_D = 256
_B = 4096
_K = 99
_KP = 128
_MOM = 0.99
_MARGIN = 10.0

_NW = 32
_BPW = _B // _NW
_RCH = 32
_TM = 20000
_NT = _M // _TM


def _sc_gather_body(mem_h, labels_h, idx_h, rows_o, bl_o, win_o,
                    idx_all, tag, rows_buf, bl_buf, win_buf, sem):
    c = lax.axis_index("c")
    s = lax.axis_index("s")
    wid = s * 2 + c
    base = wid * _BPW

    @pl.when(wid == 0)
    def _():
        pltpu.sync_copy(idx_h, idx_all)

    @pl.when(wid != 0)
    def _():
        pltpu.sync_copy(idx_h.at[pl.ds(base, _BPW)],
                        idx_all.at[pl.ds(base, _BPW)])

    for r in range(_BPW // _RCH):
        off = base + r * _RCH
        pltpu.async_copy(mem_h.at[idx_all.at[pl.ds(off, _RCH)]],
                         rows_buf, sem).wait()
        pltpu.sync_copy(rows_buf, rows_o.at[pl.ds(off, _RCH)])

    pltpu.async_copy(labels_h.at[idx_all.at[pl.ds(base, _BPW)]],
                     bl_buf, sem).wait()
    pltpu.sync_copy(bl_buf, bl_o.at[pl.ds(base, _BPW)])

    @pl.when(wid == 0)
    def _():
        def scat(j, carry):
            iv = idx_all[pl.ds(j * 16, 16)]
            jv = lax.iota(jnp.int32, 16) + j * 16
            plsc.store_scatter(tag, (iv,), jv)
            return carry

        lax.fori_loop(0, _B // 16, scat, 0)

        def gath(j, carry):
            iv = idx_all[pl.ds(j * 16, 16)]
            jv = lax.iota(jnp.int32, 16) + j * 16
            t = plsc.load_gather(tag, (iv,))
            win_buf[pl.ds(j * 16, 16)] = jnp.where(t == jv, 1.0, 0.0)
            return carry

        lax.fori_loop(0, _B // 16, gath, 0)
        pltpu.sync_copy(win_buf, win_o)


def _sc_gather(mem, bank_labels, idx):
    mesh = plsc.VectorSubcoreMesh(core_axis_name="c", subcore_axis_name="s")
    f = pl.kernel(
        _sc_gather_body,
        out_type=[
            jax.ShapeDtypeStruct((_B, _D), jnp.float32),
            jax.ShapeDtypeStruct((_B,), jnp.int32),
            jax.ShapeDtypeStruct((_B,), jnp.float32),
        ],
        mesh=mesh,
        scratch_types=[
            pltpu.VMEM((_B,), jnp.int32),
            pltpu.VMEM((_M,), jnp.int32),
            pltpu.VMEM((_RCH, _D), jnp.float32),
            pltpu.VMEM((_BPW,), jnp.int32),
            pltpu.VMEM((_B,), jnp.float32),
            pltpu.SemaphoreType.DMA,
        ],
        compiler_params=pltpu.CompilerParams(needs_layout_passes=False),
    )
    return f(mem, bank_labels, idx)


def _tc_sweep_body(lab_ref, mem_ref, sums_ref, counts_ref):
    t = pl.program_id(0)

    @pl.when(t == 0)
    def _():
        sums_ref[...] = jnp.zeros_like(sums_ref)
        counts_ref[...] = jnp.zeros_like(counts_ref)

    lab = lab_ref[0].astype(jnp.bfloat16)
    kio = lax.broadcasted_iota(
        jnp.int32, (_KP, _TM), 0).astype(jnp.bfloat16)
    eq = lab == kio
    oh = jnp.where(eq, jnp.bfloat16(1.0), jnp.bfloat16(0.0))
    sums_ref[...] += jnp.dot(oh, mem_ref[...],
                             preferred_element_type=jnp.float32)
    counts_ref[...] += jnp.sum(oh, axis=1, keepdims=True,
                               dtype=jnp.float32)


def _tc_sweep(labels3, mem):
    return pl.pallas_call(
        _tc_sweep_body,
        grid=(_NT,),
        in_specs=[
            pl.BlockSpec((1, 1, _TM), lambda i: (i, 0, 0)),
            pl.BlockSpec((_TM, _D), lambda i: (i, 0)),
        ],
        out_specs=[
            pl.BlockSpec((_KP, _D), lambda i: (0, 0)),
            pl.BlockSpec((_KP, 1), lambda i: (0, 0)),
        ],
        out_shape=[
            jax.ShapeDtypeStruct((_KP, _D), jnp.float32),
            jax.ShapeDtypeStruct((_KP, 1), jnp.float32),
        ],
    )(labels3, mem)


def _tc_epilogue_body(sums_ref, counts_ref, rows_ref, blg_ref, win_ref,
                      ilr_ref, x_ref, proto_ref, out_ref):
    kiob = lax.broadcasted_iota(jnp.int32, (_KP, _B), 0)
    win = win_ref[...]
    oh_sub = jnp.where(blg_ref[...] == kiob, win, 0.0)
    oh_add = jnp.where(ilr_ref[...] == kiob, win, 0.0)
    x = x_ref[...]
    s2 = (sums_ref[...]
          + jnp.dot(oh_add, x, preferred_element_type=jnp.float32)
          - jnp.dot(oh_sub, rows_ref[...],
                    preferred_element_type=jnp.float32))
    c2 = counts_ref[...] + jnp.sum(oh_add - oh_sub, axis=1, keepdims=True)

    proto_pad = jnp.concatenate(
        [proto_ref[...], jnp.zeros((_KP - _K, _D), jnp.float32)], axis=0)
    proto = _MOM * proto_pad + (1.0 - _MOM) * (s2 / jnp.maximum(c2, 1.0))

    xp = lax.dot_general(proto, x, (((1,), (1,)), ((), ())),
                         preferred_element_type=jnp.float32)
    ones_row = jnp.ones((1, _D), dtype=jnp.float32)
    x_sq = lax.dot_general(ones_row, x * x, (((1,), (1,)), ((), ())),
                           preferred_element_type=jnp.float32)
    p_sq = jnp.sum(proto * proto, axis=1, keepdims=True)
    d2 = p_sq + x_sq - 2.0 * xp
    d = jnp.sqrt(jnp.maximum(d2, 1e-12))
    own = ilr_ref[...] == kiob
    per = jnp.where(own, d, jnp.maximum(0.0, _MARGIN - d))
    per = jnp.where(kiob < _K, per, 0.0)
    out_ref[...] = (jnp.sum(per) / (_B * _K)) * jnp.ones((1, 1), jnp.float32)


def _tc_epilogue(sums, counts, rows, blg_r, win_r, il_r, x, proto):
    return pl.pallas_call(
        _tc_epilogue_body,
        out_shape=jax.ShapeDtypeStruct((1, 1), jnp.float32),
    )(sums, counts, rows, blg_r, win_r, il_r, x, proto)


def kernel(mem, bank_labels, idx, input_vectors, input_labels, prototypes):
    rows, blg, win = _sc_gather(mem, bank_labels, idx)
    labels3 = bank_labels.reshape(_NT, 1, _TM)
    sums, counts = _tc_sweep(labels3, mem)
    blg_r = blg.reshape(1, _B)
    win_r = win.reshape(1, _B)
    il_r = input_labels.reshape(1, _B)
    out = _tc_epilogue(sums, counts, rows, blg_r, win_r, il_r,
                       input_vectors, prototypes)
    return out[0, 0]


# speedup vs baseline: 1.1111x; 1.0283x over previous
import jax
import jax.numpy as jnp
from jax import lax
from jax.experimental import pallas as pl
from jax.experimental.pallas import tpu as pltpu
from jax.experimental.pallas import tpu_sc as plsc

_M = 100000
_D = 256
_B = 4096
_K = 99
_KP = 128
_MOM = 0.99
_MARGIN = 10.0

_NW = 32
_BPW = _B // _NW
_RCH = 32
_TM = 10000
_NT = _M // _TM


def _sc_gather_body(mem_h, labels_h, idx_h, rows_o, bl_o, win_o,
                    idx_all, tag, rows_buf, bl_buf, win_buf, sem):
    c = lax.axis_index("c")
    s = lax.axis_index("s")
    wid = s * 2 + c
    base = wid * _BPW

    @pl.when(wid == 0)
    def _():
        pltpu.sync_copy(idx_h, idx_all)

    @pl.when(wid != 0)
    def _():
        pltpu.sync_copy(idx_h.at[pl.ds(base, _BPW)],
                        idx_all.at[pl.ds(base, _BPW)])

    for r in range(_BPW // _RCH):
        off = base + r * _RCH
        pltpu.async_copy(mem_h.at[idx_all.at[pl.ds(off, _RCH)]],
                         rows_buf, sem).wait()
        pltpu.sync_copy(rows_buf, rows_o.at[pl.ds(off, _RCH)])

    pltpu.async_copy(labels_h.at[idx_all.at[pl.ds(base, _BPW)]],
                     bl_buf, sem).wait()
    pltpu.sync_copy(bl_buf, bl_o.at[pl.ds(base, _BPW)])

    @pl.when(wid == 0)
    def _():
        def scat(j, carry):
            iv = idx_all[pl.ds(j * 16, 16)]
            jv = lax.iota(jnp.int32, 16) + j * 16
            plsc.store_scatter(tag, (iv,), jv)
            return carry

        lax.fori_loop(0, _B // 16, scat, 0)

        def gath(j, carry):
            iv = idx_all[pl.ds(j * 16, 16)]
            jv = lax.iota(jnp.int32, 16) + j * 16
            t = plsc.load_gather(tag, (iv,))
            win_buf[pl.ds(j * 16, 16)] = jnp.where(t == jv, 1.0, 0.0)
            return carry

        lax.fori_loop(0, _B // 16, gath, 0)
        pltpu.sync_copy(win_buf, win_o)


def _sc_gather(mem, bank_labels, idx):
    mesh = plsc.VectorSubcoreMesh(core_axis_name="c", subcore_axis_name="s")
    f = pl.kernel(
        _sc_gather_body,
        out_type=[
            jax.ShapeDtypeStruct((_B, _D), jnp.float32),
            jax.ShapeDtypeStruct((_B,), jnp.int32),
            jax.ShapeDtypeStruct((_B,), jnp.float32),
        ],
        mesh=mesh,
        scratch_types=[
            pltpu.VMEM((_B,), jnp.int32),
            pltpu.VMEM((_M,), jnp.int32),
            pltpu.VMEM((_RCH, _D), jnp.float32),
            pltpu.VMEM((_BPW,), jnp.int32),
            pltpu.VMEM((_B,), jnp.float32),
            pltpu.SemaphoreType.DMA,
        ],
        compiler_params=pltpu.CompilerParams(needs_layout_passes=False),
    )
    return f(mem, bank_labels, idx)


def _tc_sweep_body(lab_ref, mem_ref, sums_ref, counts_ref):
    t = pl.program_id(0)

    @pl.when(t == 0)
    def _():
        sums_ref[...] = jnp.zeros_like(sums_ref)
        counts_ref[...] = jnp.zeros_like(counts_ref)

    lab = lab_ref[0].astype(jnp.bfloat16)
    kio = lax.broadcasted_iota(
        jnp.int32, (_KP, _TM), 0).astype(jnp.bfloat16)
    eq = lab == kio
    oh = jnp.where(eq, jnp.bfloat16(1.0), jnp.bfloat16(0.0))
    sums_ref[...] += jnp.dot(oh, mem_ref[...],
                             preferred_element_type=jnp.float32)
    counts_ref[...] += jnp.sum(oh, axis=1, keepdims=True,
                               dtype=jnp.float32)


def _tc_sweep(labels3, mem):
    return pl.pallas_call(
        _tc_sweep_body,
        grid=(_NT,),
        in_specs=[
            pl.BlockSpec((1, 1, _TM), lambda i: (i, 0, 0)),
            pl.BlockSpec((_TM, _D), lambda i: (i, 0)),
        ],
        out_specs=[
            pl.BlockSpec((_KP, _D), lambda i: (0, 0)),
            pl.BlockSpec((_KP, 1), lambda i: (0, 0)),
        ],
        out_shape=[
            jax.ShapeDtypeStruct((_KP, _D), jnp.float32),
            jax.ShapeDtypeStruct((_KP, 1), jnp.float32),
        ],
    )(labels3, mem)


def _tc_epilogue_body(sums_ref, counts_ref, rows_ref, blg_ref, win_ref,
                      ilr_ref, x_ref, proto_ref, out_ref):
    kiob = lax.broadcasted_iota(jnp.int32, (_KP, _B), 0)
    win = win_ref[...]
    oh_sub = jnp.where(blg_ref[...] == kiob, win, 0.0)
    oh_add = jnp.where(ilr_ref[...] == kiob, win, 0.0)
    x = x_ref[...]
    s2 = (sums_ref[...]
          + jnp.dot(oh_add, x, preferred_element_type=jnp.float32)
          - jnp.dot(oh_sub, rows_ref[...],
                    preferred_element_type=jnp.float32))
    c2 = counts_ref[...] + jnp.sum(oh_add - oh_sub, axis=1, keepdims=True)

    proto_pad = jnp.concatenate(
        [proto_ref[...], jnp.zeros((_KP - _K, _D), jnp.float32)], axis=0)
    proto = _MOM * proto_pad + (1.0 - _MOM) * (s2 / jnp.maximum(c2, 1.0))

    xp = lax.dot_general(proto, x, (((1,), (1,)), ((), ())),
                         preferred_element_type=jnp.float32)
    ones_row = jnp.ones((1, _D), dtype=jnp.float32)
    x_sq = lax.dot_general(ones_row, x * x, (((1,), (1,)), ((), ())),
                           preferred_element_type=jnp.float32)
    p_sq = jnp.sum(proto * proto, axis=1, keepdims=True)
    d2 = p_sq + x_sq - 2.0 * xp
    d = jnp.sqrt(jnp.maximum(d2, 1e-12))
    own = ilr_ref[...] == kiob
    per = jnp.where(own, d, jnp.maximum(0.0, _MARGIN - d))
    per = jnp.where(kiob < _K, per, 0.0)
    out_ref[...] = (jnp.sum(per) / (_B * _K)) * jnp.ones((1, 1), jnp.float32)


def _tc_epilogue(sums, counts, rows, blg_r, win_r, il_r, x, proto):
    return pl.pallas_call(
        _tc_epilogue_body,
        out_shape=jax.ShapeDtypeStruct((1, 1), jnp.float32),
    )(sums, counts, rows, blg_r, win_r, il_r, x, proto)


def kernel(mem, bank_labels, idx, input_vectors, input_labels, prototypes):
    rows, blg, win = _sc_gather(mem, bank_labels, idx)
    labels3 = bank_labels.reshape(_NT, 1, _TM)
    sums, counts = _tc_sweep(labels3, mem)
    blg_r = blg.reshape(1, _B)
    win_r = win.reshape(1, _B)
    il_r = input_labels.reshape(1, _B)
    out = _tc_epilogue(sums, counts, rows, blg_r, win_r, il_r,
                       input_vectors, prototypes)
    return out[0, 0]
